# baseline (speedup 1.0000x reference)
import functools

import jax
import jax.numpy as jnp
from jax import lax
from jax.experimental import pallas as pl
from jax.experimental.pallas import tpu as pltpu
from jax.experimental.pallas import tpu_sc as plsc

N = 10000
D = 128
E = 320000

NC = 2
NS = 16
NW = NC * NS

NROW = 10240
SCRAP = 10100

CH = 128
CPT = 80
CPT0 = 56
CPT1 = 2 * CPT - CPT0
CPTMX = max(CPT0, CPT1)
EPAD = NW * CPT * CH

_MESH = plsc.VectorSubcoreMesh(core_axis_name="c", subcore_axis_name="s")


@functools.partial(
    pl.kernel,
    mesh=_MESH,
    out_type=jax.ShapeDtypeStruct((NC, NROW, D), jnp.float32),
    scratch_types=[
        pltpu.VMEM((CH, D), jnp.float32),
        pltpu.VMEM((CH, D), jnp.float32),
        pltpu.VMEM((CPT, CH), jnp.int32),
        pltpu.VMEM_SHARED((NROW, D), jnp.float32),
        pltpu.SemaphoreType.DMA,
    ],
)
def _hist_kernel(idx_hbm, out_hbm, zbuf_v, obuf_v, hidx_v, tab_s, sem):
    core = lax.axis_index("c")
    sub = lax.axis_index("s")
    tid = core * NS + sub

    def fillz(r, c):
        for j in range(D // 16):
            zbuf_v[r, pl.ds(j * 16, 16)] = jnp.zeros((16,), jnp.float32)
            obuf_v[r, pl.ds(j * 16, 16)] = jnp.full((16,), 1.0, jnp.float32)
        return c
    lax.fori_loop(0, CH, fillz, None)
    pltpu.async_copy(idx_hbm.at[pl.ds(tid * CPT, CPT)], hidx_v, sem)
    zrows = NROW // NS

    def zbody(k, c):
        pltpu.sync_copy(zbuf_v, tab_s.at[pl.ds(sub * zrows + k * CH, CH)])
        return c
    lax.fori_loop(0, zrows // CH, zbody, None)
    pltpu.make_async_copy(idx_hbm.at[pl.ds(tid * CPT, CPT)], hidx_v, sem).wait()
    plsc.subcore_barrier()

    def chunk(k, c):
        pltpu.sync_copy(obuf_v, tab_s.at[hidx_v.at[k]], add=True)
        return c
    lax.fori_loop(0, CPT, chunk, None)
    plsc.subcore_barrier()

    pltpu.sync_copy(
        tab_s.at[pl.ds(sub * zrows, zrows)],
        out_hbm.at[core, pl.ds(sub * zrows, zrows)],
    )


@functools.partial(
    pl.kernel,
    mesh=_MESH,
    out_type=jax.ShapeDtypeStruct((NC, NROW, D), jnp.float32),
    scratch_types=[
        pltpu.VMEM((CH, D), jnp.float32),
        pltpu.VMEM((CH, D), jnp.float32),
        pltpu.VMEM((CPTMX, CH), jnp.int32),
        pltpu.VMEM((CH,), jnp.int32),
        pltpu.VMEM((CH,), jnp.int32),
        pltpu.VMEM_SHARED((NROW, D), jnp.float32),
        pltpu.SemaphoreType.DMA,
        pltpu.SemaphoreType.DMA,
        pltpu.SemaphoreType.DMA,
        pltpu.SemaphoreType.DMA,
    ],
)
def _scatter_kernel(hp_hbm, src_hbm, dst_hbm, out_hbm,
                    rows0_v, rows1_v, sidx_v, d0_v, d1_v, acc_s,
                    sem0, sem1, dsem0, dsem1):
    core = lax.axis_index("c")
    sub = lax.axis_index("s")
    my_cpt = jnp.where(core == 0, CPT0, CPT1)
    cbase = jnp.where(core == 0, sub * CPT0, NS * CPT0 + sub * CPT1)

    pltpu.async_copy(src_hbm.at[pl.ds(cbase, CPTMX)], sidx_v, sem0)

    def zfill(r, c):
        for j in range(D // 16):
            rows0_v[r, pl.ds(j * 16, 16)] = jnp.zeros((16,), jnp.float32)
        return c
    lax.fori_loop(0, CH, zfill, None)
    zrows = NROW // NS

    def zbody(k, c):
        pltpu.sync_copy(rows0_v, acc_s.at[pl.ds(sub * zrows + k * CH, CH)])
        return c
    lax.fori_loop(0, zrows // CH, zbody, None)
    pltpu.make_async_copy(src_hbm.at[pl.ds(cbase, CPTMX)], sidx_v, sem0).wait()
    plsc.subcore_barrier()

    dbase = cbase
    half = my_cpt // 2
    pltpu.async_copy(hp_hbm.at[sidx_v.at[0]], rows0_v, sem0)
    pltpu.async_copy(dst_hbm.at[dbase], d0_v, dsem0)
    pltpu.async_copy(dst_hbm.at[dbase + 1], d1_v, dsem1)

    def pipe(k, c):
        last = k >= half - 1

        pltpu.make_async_copy(hp_hbm.at[pl.ds(0, CH)], rows0_v, sem0).wait()
        pltpu.async_copy(hp_hbm.at[sidx_v.at[2 * k + 1]], rows1_v, sem1)
        pltpu.make_async_copy(dst_hbm.at[dbase], d0_v, dsem0).wait()
        pltpu.sync_copy(rows0_v, acc_s.at[d0_v], add=True)

        pltpu.make_async_copy(hp_hbm.at[pl.ds(0, CH)], rows1_v, sem1).wait()

        @pl.when(jnp.logical_not(last))
        def _():
            pltpu.async_copy(hp_hbm.at[sidx_v.at[2 * k + 2]], rows0_v, sem0)
            pltpu.async_copy(dst_hbm.at[dbase + 2 * k + 2], d0_v, dsem0)
        pltpu.make_async_copy(dst_hbm.at[dbase], d1_v, dsem1).wait()
        pltpu.sync_copy(rows1_v, acc_s.at[d1_v], add=True)

        @pl.when(jnp.logical_not(last))
        def _():
            pltpu.async_copy(dst_hbm.at[dbase + 2 * k + 3], d1_v, dsem1)
        return c
    lax.fori_loop(0, half, pipe, None)
    plsc.subcore_barrier()

    pltpu.sync_copy(
        acc_s.at[pl.ds(sub * zrows, zrows)],
        out_hbm.at[core, pl.ds(sub * zrows, zrows)],
    )


_BLK = 1000
_GRID = N // _BLK

_row_spec = pl.BlockSpec((_BLK, D), lambda i: (i, 0))
_cnt_spec = pl.BlockSpec((_BLK, 1), lambda i: (i, 0))
_mat_spec = pl.BlockSpec((D, D), lambda i: (0, 0))
_bias_spec = pl.BlockSpec((1, D), lambda i: (0, 0))


def _tc1_body(x_ref, w_ref, ca_ref, cb_ref, o_ref):
    dis = lax.rsqrt(1.0 + ca_ref[...] + cb_ref[...])
    o_ref[...] = jnp.dot(x_ref[...], w_ref[...],
                         preferred_element_type=jnp.float32) * dis


def _tc1(x, w1, c1a, c1b):
    return pl.pallas_call(
        _tc1_body,
        grid=(_GRID,),
        in_specs=[_row_spec, _mat_spec, _cnt_spec, _cnt_spec],
        out_specs=_row_spec,
        out_shape=jax.ShapeDtypeStruct((N, D), jnp.float32),
    )(x, w1, c1a, c1b)


def _tc2_body(p0_ref, p1_ref, hp_ref, c1a_ref, c1b_ref, bias_ref,
              w2_ref, c0a_ref, c0b_ref, o_ref):
    dis1 = lax.rsqrt(1.0 + c1a_ref[...] + c1b_ref[...])
    h = dis1 * (p0_ref[...] + p1_ref[...] + hp_ref[...]) + bias_ref[...]
    h = jnp.maximum(h, 0.0)
    dis0 = lax.rsqrt(1.0 + c0a_ref[...] + c0b_ref[...])
    o_ref[...] = jnp.dot(h, w2_ref[...],
                         preferred_element_type=jnp.float32) * dis0


def _tc2(p0, p1, hp, c1a, c1b, b1, w2, c0a, c0b):
    return pl.pallas_call(
        _tc2_body,
        grid=(_GRID,),
        in_specs=[_row_spec, _row_spec, _row_spec, _cnt_spec, _cnt_spec,
                  _bias_spec, _mat_spec, _cnt_spec, _cnt_spec],
        out_specs=_row_spec,
        out_shape=jax.ShapeDtypeStruct((N, D), jnp.float32),
    )(p0, p1, hp, c1a, c1b, b1, w2, c0a, c0b)


def _tc3_body(p0_ref, p1_ref, hp_ref, ca_ref, cb_ref, bias_ref, o_ref):
    dis = lax.rsqrt(1.0 + ca_ref[...] + cb_ref[...])
    o_ref[...] = dis * (p0_ref[...] + p1_ref[...] + hp_ref[...]) + bias_ref[...]


def _tc3(p0, p1, hp, c0a, c0b, b2):
    return pl.pallas_call(
        _tc3_body,
        grid=(_GRID,),
        in_specs=[_row_spec, _row_spec, _row_spec, _cnt_spec, _cnt_spec,
                  _bias_spec],
        out_specs=_row_spec,
        out_shape=jax.ShapeDtypeStruct((N, D), jnp.float32),
    )(p0, p1, hp, c0a, c0b, b2)


def _pad_edges(src, dst):
    pad = EPAD + CPTMX * CH - E
    srcp = jnp.concatenate([src, jnp.zeros((pad,), jnp.int32)])
    dstp = jnp.concatenate([dst, jnp.full((pad,), SCRAP, jnp.int32)])
    return (srcp.reshape(NW * CPT + CPTMX, CH),
            dstp.reshape(NW * CPT + CPTMX, CH))


@jax.jit
def kernel(x, edge_index0, edge_index1, W1, b1, W2, b2):
    ei0 = edge_index0.astype(jnp.int32)
    ei1 = edge_index1.astype(jnp.int32)
    src1, dst1 = ei1[0], ei1[1]
    src0, dst0 = ei0[0], ei0[1]

    srcp1, dstp1 = _pad_edges(src1, dst1)
    srcp0, dstp0 = _pad_edges(src0, dst0)

    hist1 = _hist_kernel(dstp1)
    hist0 = _hist_kernel(dstp0)
    c1a = hist1[0, :, 0:1]
    c1b = hist1[1, :, 0:1]
    c0a = hist0[0, :, 0:1]
    c0b = hist0[1, :, 0:1]

    hp1 = _tc1(x, W1, c1a, c1b)

    acc1 = _scatter_kernel(hp1, srcp1, dstp1)

    hp2 = _tc2(acc1[0, :N], acc1[1, :N], hp1, c1a, c1b,
               b1.reshape(1, D), W2, c0a, c0b)

    acc2 = _scatter_kernel(hp2, srcp0, dstp0)

    return _tc3(acc2[0, :N], acc2[1, :N], hp2, c0a, c0b, b2.reshape(1, D))


# speedup vs baseline: 1.1239x; 1.1239x over previous
import functools

import jax
import jax.numpy as jnp
from jax import lax
from jax.experimental import pallas as pl
from jax.experimental.pallas import tpu as pltpu
from jax.experimental.pallas import tpu_sc as plsc

N = 10000
D = 128
E = 320000

NC = 2
NS = 16
NW = NC * NS

NROW = 10240
SCRAP = 10100

CH = 128
CPT = 80
CPT0 = 104
CPT1 = 2 * CPT - CPT0
CPTMX = max(CPT0, CPT1)
EPAD = NW * CPT * CH

_MESH = plsc.VectorSubcoreMesh(core_axis_name="c", subcore_axis_name="s")


@functools.partial(
    pl.kernel,
    mesh=_MESH,
    out_type=jax.ShapeDtypeStruct((NC, NROW, D), jnp.float32),
    scratch_types=[
        pltpu.VMEM((CH, D), jnp.float32),
        pltpu.VMEM((CH, D), jnp.float32),
        pltpu.VMEM((CPT, CH), jnp.int32),
        pltpu.VMEM_SHARED((NROW, D), jnp.float32),
        pltpu.SemaphoreType.DMA,
    ],
)
def _hist_kernel(idx_hbm, out_hbm, zbuf_v, obuf_v, hidx_v, tab_s, sem):
    core = lax.axis_index("c")
    sub = lax.axis_index("s")
    tid = core * NS + sub

    def fillz(r, c):
        for j in range(D // 16):
            zbuf_v[r, pl.ds(j * 16, 16)] = jnp.zeros((16,), jnp.float32)
            obuf_v[r, pl.ds(j * 16, 16)] = jnp.full((16,), 1.0, jnp.float32)
        return c
    lax.fori_loop(0, CH, fillz, None)
    pltpu.async_copy(idx_hbm.at[pl.ds(tid * CPT, CPT)], hidx_v, sem)
    zrows = NROW // NS

    def zbody(k, c):
        pltpu.sync_copy(zbuf_v, tab_s.at[pl.ds(sub * zrows + k * CH, CH)])
        return c
    lax.fori_loop(0, zrows // CH, zbody, None)
    pltpu.make_async_copy(idx_hbm.at[pl.ds(tid * CPT, CPT)], hidx_v, sem).wait()
    plsc.subcore_barrier()

    def chunk(k, c):
        pltpu.sync_copy(obuf_v, tab_s.at[hidx_v.at[k]], add=True)
        return c
    lax.fori_loop(0, CPT, chunk, None)
    plsc.subcore_barrier()

    pltpu.sync_copy(
        tab_s.at[pl.ds(sub * zrows, zrows)],
        out_hbm.at[core, pl.ds(sub * zrows, zrows)],
    )


@functools.partial(
    pl.kernel,
    mesh=_MESH,
    out_type=jax.ShapeDtypeStruct((NC, NROW, D), jnp.float32),
    scratch_types=[
        pltpu.VMEM((CH, D), jnp.float32),
        pltpu.VMEM((CH, D), jnp.float32),
        pltpu.VMEM((CPTMX, CH), jnp.int32),
        pltpu.VMEM((CH,), jnp.int32),
        pltpu.VMEM((CH,), jnp.int32),
        pltpu.VMEM_SHARED((NROW, D), jnp.float32),
        pltpu.SemaphoreType.DMA,
        pltpu.SemaphoreType.DMA,
        pltpu.SemaphoreType.DMA,
        pltpu.SemaphoreType.DMA,
    ],
)
def _scatter_kernel(hp_hbm, src_hbm, dst_hbm, out_hbm,
                    rows0_v, rows1_v, sidx_v, d0_v, d1_v, acc_s,
                    sem0, sem1, dsem0, dsem1):
    core = lax.axis_index("c")
    sub = lax.axis_index("s")
    my_cpt = jnp.where(core == 0, CPT0, CPT1)
    cbase = jnp.where(core == 0, sub * CPT0, NS * CPT0 + sub * CPT1)

    pltpu.async_copy(src_hbm.at[pl.ds(cbase, CPTMX)], sidx_v, sem0)

    def zfill(r, c):
        for j in range(D // 16):
            rows0_v[r, pl.ds(j * 16, 16)] = jnp.zeros((16,), jnp.float32)
        return c
    lax.fori_loop(0, CH, zfill, None)
    zrows = NROW // NS

    def zbody(k, c):
        pltpu.sync_copy(rows0_v, acc_s.at[pl.ds(sub * zrows + k * CH, CH)])
        return c
    lax.fori_loop(0, zrows // CH, zbody, None)
    pltpu.make_async_copy(src_hbm.at[pl.ds(cbase, CPTMX)], sidx_v, sem0).wait()
    plsc.subcore_barrier()

    dbase = cbase
    half = my_cpt // 2
    pltpu.async_copy(hp_hbm.at[sidx_v.at[0]], rows0_v, sem0)
    pltpu.async_copy(dst_hbm.at[dbase], d0_v, dsem0)
    pltpu.async_copy(dst_hbm.at[dbase + 1], d1_v, dsem1)

    def pipe(k, c):
        last = k >= half - 1

        pltpu.make_async_copy(hp_hbm.at[pl.ds(0, CH)], rows0_v, sem0).wait()
        pltpu.async_copy(hp_hbm.at[sidx_v.at[2 * k + 1]], rows1_v, sem1)
        pltpu.make_async_copy(dst_hbm.at[dbase], d0_v, dsem0).wait()
        pltpu.sync_copy(rows0_v, acc_s.at[d0_v], add=True)

        pltpu.make_async_copy(hp_hbm.at[pl.ds(0, CH)], rows1_v, sem1).wait()

        @pl.when(jnp.logical_not(last))
        def _():
            pltpu.async_copy(hp_hbm.at[sidx_v.at[2 * k + 2]], rows0_v, sem0)
            pltpu.async_copy(dst_hbm.at[dbase + 2 * k + 2], d0_v, dsem0)
        pltpu.make_async_copy(dst_hbm.at[dbase], d1_v, dsem1).wait()
        pltpu.sync_copy(rows1_v, acc_s.at[d1_v], add=True)

        @pl.when(jnp.logical_not(last))
        def _():
            pltpu.async_copy(dst_hbm.at[dbase + 2 * k + 3], d1_v, dsem1)
        return c
    lax.fori_loop(0, half, pipe, None)
    plsc.subcore_barrier()

    pltpu.sync_copy(
        acc_s.at[pl.ds(sub * zrows, zrows)],
        out_hbm.at[core, pl.ds(sub * zrows, zrows)],
    )


_BLK = 1000
_GRID = N // _BLK

_row_spec = pl.BlockSpec((_BLK, D), lambda i: (i, 0))
_cnt_spec = pl.BlockSpec((_BLK, 1), lambda i: (i, 0))
_mat_spec = pl.BlockSpec((D, D), lambda i: (0, 0))
_bias_spec = pl.BlockSpec((1, D), lambda i: (0, 0))


def _tc1_body(x_ref, w_ref, ca_ref, cb_ref, o_ref):
    dis = lax.rsqrt(1.0 + ca_ref[...] + cb_ref[...])
    o_ref[...] = jnp.dot(x_ref[...], w_ref[...],
                         preferred_element_type=jnp.float32) * dis


def _tc1(x, w1, c1a, c1b):
    return pl.pallas_call(
        _tc1_body,
        grid=(_GRID,),
        in_specs=[_row_spec, _mat_spec, _cnt_spec, _cnt_spec],
        out_specs=_row_spec,
        out_shape=jax.ShapeDtypeStruct((N, D), jnp.float32),
    )(x, w1, c1a, c1b)


def _tc2_body(p0_ref, p1_ref, hp_ref, c1a_ref, c1b_ref, bias_ref,
              w2_ref, c0a_ref, c0b_ref, o_ref):
    dis1 = lax.rsqrt(1.0 + c1a_ref[...] + c1b_ref[...])
    h = dis1 * (p0_ref[...] + p1_ref[...] + hp_ref[...]) + bias_ref[...]
    h = jnp.maximum(h, 0.0)
    dis0 = lax.rsqrt(1.0 + c0a_ref[...] + c0b_ref[...])
    o_ref[...] = jnp.dot(h, w2_ref[...],
                         preferred_element_type=jnp.float32) * dis0


def _tc2(p0, p1, hp, c1a, c1b, b1, w2, c0a, c0b):
    return pl.pallas_call(
        _tc2_body,
        grid=(_GRID,),
        in_specs=[_row_spec, _row_spec, _row_spec, _cnt_spec, _cnt_spec,
                  _bias_spec, _mat_spec, _cnt_spec, _cnt_spec],
        out_specs=_row_spec,
        out_shape=jax.ShapeDtypeStruct((N, D), jnp.float32),
    )(p0, p1, hp, c1a, c1b, b1, w2, c0a, c0b)


def _tc3_body(p0_ref, p1_ref, hp_ref, ca_ref, cb_ref, bias_ref, o_ref):
    dis = lax.rsqrt(1.0 + ca_ref[...] + cb_ref[...])
    o_ref[...] = dis * (p0_ref[...] + p1_ref[...] + hp_ref[...]) + bias_ref[...]


def _tc3(p0, p1, hp, c0a, c0b, b2):
    return pl.pallas_call(
        _tc3_body,
        grid=(_GRID,),
        in_specs=[_row_spec, _row_spec, _row_spec, _cnt_spec, _cnt_spec,
                  _bias_spec],
        out_specs=_row_spec,
        out_shape=jax.ShapeDtypeStruct((N, D), jnp.float32),
    )(p0, p1, hp, c0a, c0b, b2)


def _pad_edges(src, dst):
    pad = EPAD + CPTMX * CH - E
    srcp = jnp.concatenate([src, jnp.zeros((pad,), jnp.int32)])
    dstp = jnp.concatenate([dst, jnp.full((pad,), SCRAP, jnp.int32)])
    return (srcp.reshape(NW * CPT + CPTMX, CH),
            dstp.reshape(NW * CPT + CPTMX, CH))


@jax.jit
def kernel(x, edge_index0, edge_index1, W1, b1, W2, b2):
    ei0 = edge_index0.astype(jnp.int32)
    ei1 = edge_index1.astype(jnp.int32)
    src1, dst1 = ei1[0], ei1[1]
    src0, dst0 = ei0[0], ei0[1]

    srcp1, dstp1 = _pad_edges(src1, dst1)
    srcp0, dstp0 = _pad_edges(src0, dst0)

    hist1 = _hist_kernel(dstp1)
    hist0 = _hist_kernel(dstp0)
    c1a = hist1[0, :, 0:1]
    c1b = hist1[1, :, 0:1]
    c0a = hist0[0, :, 0:1]
    c0b = hist0[1, :, 0:1]

    hp1 = _tc1(x, W1, c1a, c1b)

    acc1 = _scatter_kernel(hp1, srcp1, dstp1)

    hp2 = _tc2(acc1[0, :N], acc1[1, :N], hp1, c1a, c1b,
               b1.reshape(1, D), W2, c0a, c0b)

    acc2 = _scatter_kernel(hp2, srcp0, dstp0)

    return _tc3(acc2[0, :N], acc2[1, :N], hp2, c0a, c0b, b2.reshape(1, D))


# speedup vs baseline: 1.1702x; 1.0412x over previous
import functools

import jax
import jax.numpy as jnp
from jax import lax
from jax.experimental import pallas as pl
from jax.experimental.pallas import tpu as pltpu
from jax.experimental.pallas import tpu_sc as plsc

N = 10000
D = 128
E = 320000

NC = 2
NS = 16
NW = NC * NS

NROW = 10112
SCRAP = 10100

CH = 128
CPT = 80
CPT0 = 128
CPT1 = 2 * CPT - CPT0
CPTMX = max(CPT0, CPT1)
EPAD = NW * CPT * CH

_MESH = plsc.VectorSubcoreMesh(core_axis_name="c", subcore_axis_name="s")


@functools.partial(
    pl.kernel,
    mesh=_MESH,
    out_type=jax.ShapeDtypeStruct((NC, NROW, D), jnp.float32),
    scratch_types=[
        pltpu.VMEM((CH, D), jnp.float32),
        pltpu.VMEM((CH, D), jnp.float32),
        pltpu.VMEM((CPT, CH), jnp.int32),
        pltpu.VMEM_SHARED((NROW, D), jnp.float32),
        pltpu.SemaphoreType.DMA,
    ],
)
def _hist_kernel(idx_hbm, out_hbm, zbuf_v, obuf_v, hidx_v, tab_s, sem):
    core = lax.axis_index("c")
    sub = lax.axis_index("s")
    tid = core * NS + sub

    def fillz(r, c):
        for j in range(D // 16):
            zbuf_v[r, pl.ds(j * 16, 16)] = jnp.zeros((16,), jnp.float32)
            obuf_v[r, pl.ds(j * 16, 16)] = jnp.full((16,), 1.0, jnp.float32)
        return c
    lax.fori_loop(0, CH, fillz, None)
    pltpu.async_copy(idx_hbm.at[pl.ds(tid * CPT, CPT)], hidx_v, sem)
    zrows = NROW // NS

    def zbody(k, c):
        pltpu.sync_copy(zbuf_v, tab_s.at[pl.ds(sub * zrows + k * CH, CH)])
        return c
    lax.fori_loop(0, zrows // CH, zbody, None)
    zrem = zrows % CH
    pltpu.sync_copy(
        zbuf_v.at[pl.ds(0, zrem)],
        tab_s.at[pl.ds(sub * zrows + zrows - zrem, zrem)],
    )
    pltpu.make_async_copy(idx_hbm.at[pl.ds(tid * CPT, CPT)], hidx_v, sem).wait()
    plsc.subcore_barrier()

    def chunk(k, c):
        pltpu.sync_copy(obuf_v, tab_s.at[hidx_v.at[k]], add=True)
        return c
    lax.fori_loop(0, CPT, chunk, None)
    plsc.subcore_barrier()

    pltpu.sync_copy(
        tab_s.at[pl.ds(sub * zrows, zrows)],
        out_hbm.at[core, pl.ds(sub * zrows, zrows)],
    )


@functools.partial(
    pl.kernel,
    mesh=_MESH,
    out_type=jax.ShapeDtypeStruct((NC, NROW, D), jnp.float32),
    scratch_types=[
        pltpu.VMEM((CH, D), jnp.float32),
        pltpu.VMEM((CH, D), jnp.float32),
        pltpu.VMEM((CPTMX, CH), jnp.int32),
        pltpu.VMEM((CH,), jnp.int32),
        pltpu.VMEM((CH,), jnp.int32),
        pltpu.VMEM_SHARED((NROW, D), jnp.float32),
        pltpu.SemaphoreType.DMA,
        pltpu.SemaphoreType.DMA,
        pltpu.SemaphoreType.DMA,
        pltpu.SemaphoreType.DMA,
    ],
)
def _scatter_kernel(hp_hbm, src_hbm, dst_hbm, out_hbm,
                    rows0_v, rows1_v, sidx_v, d0_v, d1_v, acc_s,
                    sem0, sem1, dsem0, dsem1):
    core = lax.axis_index("c")
    sub = lax.axis_index("s")
    my_cpt = jnp.where(core == 0, CPT0, CPT1)
    cbase = jnp.where(core == 0, sub * CPT0, NS * CPT0 + sub * CPT1)

    pltpu.async_copy(src_hbm.at[pl.ds(cbase, CPTMX)], sidx_v, sem0)

    def zfill(r, c):
        for j in range(D // 16):
            rows0_v[r, pl.ds(j * 16, 16)] = jnp.zeros((16,), jnp.float32)
        return c
    lax.fori_loop(0, CH, zfill, None)
    zrows = NROW // NS

    def zbody(k, c):
        pltpu.sync_copy(rows0_v, acc_s.at[pl.ds(sub * zrows + k * CH, CH)])
        return c
    lax.fori_loop(0, zrows // CH, zbody, None)
    zrem = zrows % CH
    pltpu.sync_copy(
        rows0_v.at[pl.ds(0, zrem)],
        acc_s.at[pl.ds(sub * zrows + zrows - zrem, zrem)],
    )
    pltpu.make_async_copy(src_hbm.at[pl.ds(cbase, CPTMX)], sidx_v, sem0).wait()
    plsc.subcore_barrier()

    dbase = cbase
    half = my_cpt // 2

    @pl.when(half > 0)
    def _():
        pltpu.async_copy(hp_hbm.at[sidx_v.at[0]], rows0_v, sem0)
        pltpu.async_copy(dst_hbm.at[dbase], d0_v, dsem0)
        pltpu.async_copy(dst_hbm.at[dbase + 1], d1_v, dsem1)

    def pipe(k, c):
        last = k >= half - 1

        pltpu.make_async_copy(hp_hbm.at[pl.ds(0, CH)], rows0_v, sem0).wait()
        pltpu.async_copy(hp_hbm.at[sidx_v.at[2 * k + 1]], rows1_v, sem1)
        pltpu.make_async_copy(dst_hbm.at[dbase], d0_v, dsem0).wait()
        pltpu.sync_copy(rows0_v, acc_s.at[d0_v], add=True)

        pltpu.make_async_copy(hp_hbm.at[pl.ds(0, CH)], rows1_v, sem1).wait()

        @pl.when(jnp.logical_not(last))
        def _():
            pltpu.async_copy(hp_hbm.at[sidx_v.at[2 * k + 2]], rows0_v, sem0)
            pltpu.async_copy(dst_hbm.at[dbase + 2 * k + 2], d0_v, dsem0)
        pltpu.make_async_copy(dst_hbm.at[dbase], d1_v, dsem1).wait()
        pltpu.sync_copy(rows1_v, acc_s.at[d1_v], add=True)

        @pl.when(jnp.logical_not(last))
        def _():
            pltpu.async_copy(dst_hbm.at[dbase + 2 * k + 3], d1_v, dsem1)
        return c
    lax.fori_loop(0, half, pipe, None)
    plsc.subcore_barrier()

    pltpu.sync_copy(
        acc_s.at[pl.ds(sub * zrows, zrows)],
        out_hbm.at[core, pl.ds(sub * zrows, zrows)],
    )


_BLK = 1000
_GRID = N // _BLK

_row_spec = pl.BlockSpec((_BLK, D), lambda i: (i, 0))
_cnt_spec = pl.BlockSpec((_BLK, 1), lambda i: (i, 0))
_mat_spec = pl.BlockSpec((D, D), lambda i: (0, 0))
_bias_spec = pl.BlockSpec((1, D), lambda i: (0, 0))


def _tc1_body(x_ref, w_ref, ca_ref, cb_ref, o_ref):
    dis = lax.rsqrt(1.0 + ca_ref[...] + cb_ref[...])
    o_ref[...] = jnp.dot(x_ref[...], w_ref[...],
                         preferred_element_type=jnp.float32) * dis


def _tc1(x, w1, c1a, c1b):
    return pl.pallas_call(
        _tc1_body,
        grid=(_GRID,),
        in_specs=[_row_spec, _mat_spec, _cnt_spec, _cnt_spec],
        out_specs=_row_spec,
        out_shape=jax.ShapeDtypeStruct((N, D), jnp.float32),
    )(x, w1, c1a, c1b)


def _tc2_body(p0_ref, p1_ref, hp_ref, c1a_ref, c1b_ref, bias_ref,
              w2_ref, c0a_ref, c0b_ref, o_ref):
    dis1 = lax.rsqrt(1.0 + c1a_ref[...] + c1b_ref[...])
    h = dis1 * (p0_ref[...] + p1_ref[...] + hp_ref[...]) + bias_ref[...]
    h = jnp.maximum(h, 0.0)
    dis0 = lax.rsqrt(1.0 + c0a_ref[...] + c0b_ref[...])
    o_ref[...] = jnp.dot(h, w2_ref[...],
                         preferred_element_type=jnp.float32) * dis0


def _tc2(p0, p1, hp, c1a, c1b, b1, w2, c0a, c0b):
    return pl.pallas_call(
        _tc2_body,
        grid=(_GRID,),
        in_specs=[_row_spec, _row_spec, _row_spec, _cnt_spec, _cnt_spec,
                  _bias_spec, _mat_spec, _cnt_spec, _cnt_spec],
        out_specs=_row_spec,
        out_shape=jax.ShapeDtypeStruct((N, D), jnp.float32),
    )(p0, p1, hp, c1a, c1b, b1, w2, c0a, c0b)


def _tc3_body(p0_ref, p1_ref, hp_ref, ca_ref, cb_ref, bias_ref, o_ref):
    dis = lax.rsqrt(1.0 + ca_ref[...] + cb_ref[...])
    o_ref[...] = dis * (p0_ref[...] + p1_ref[...] + hp_ref[...]) + bias_ref[...]


def _tc3(p0, p1, hp, c0a, c0b, b2):
    return pl.pallas_call(
        _tc3_body,
        grid=(_GRID,),
        in_specs=[_row_spec, _row_spec, _row_spec, _cnt_spec, _cnt_spec,
                  _bias_spec],
        out_specs=_row_spec,
        out_shape=jax.ShapeDtypeStruct((N, D), jnp.float32),
    )(p0, p1, hp, c0a, c0b, b2)


def _pad_edges(src, dst):
    pad = EPAD + CPTMX * CH - E
    srcp = jnp.concatenate([src, jnp.zeros((pad,), jnp.int32)])
    dstp = jnp.concatenate([dst, jnp.full((pad,), SCRAP, jnp.int32)])
    return (srcp.reshape(NW * CPT + CPTMX, CH),
            dstp.reshape(NW * CPT + CPTMX, CH))


@jax.jit
def kernel(x, edge_index0, edge_index1, W1, b1, W2, b2):
    ei0 = edge_index0.astype(jnp.int32)
    ei1 = edge_index1.astype(jnp.int32)
    src1, dst1 = ei1[0], ei1[1]
    src0, dst0 = ei0[0], ei0[1]

    srcp1, dstp1 = _pad_edges(src1, dst1)
    srcp0, dstp0 = _pad_edges(src0, dst0)

    hist1 = _hist_kernel(dstp1)
    hist0 = _hist_kernel(dstp0)
    c1a = hist1[0, :, 0:1]
    c1b = hist1[1, :, 0:1]
    c0a = hist0[0, :, 0:1]
    c0b = hist0[1, :, 0:1]

    hp1 = _tc1(x, W1, c1a, c1b)

    acc1 = _scatter_kernel(hp1, srcp1, dstp1)

    hp2 = _tc2(acc1[0, :N], acc1[1, :N], hp1, c1a, c1b,
               b1.reshape(1, D), W2, c0a, c0b)

    acc2 = _scatter_kernel(hp2, srcp0, dstp0)

    return _tc3(acc2[0, :N], acc2[1, :N], hp2, c0a, c0b, b2.reshape(1, D))
